# baseline (speedup 1.0000x reference)
import functools

import jax
import jax.numpy as jnp
from jax import lax
from jax.experimental import pallas as pl
from jax.experimental.pallas import tpu as pltpu
from jax.experimental.pallas import tpu_sc as plsc

_LANES = 16


def kernel(input, table):
    B, L = input.shape
    V = table.shape[0]
    info = plsc.get_sparse_core_info()
    nw = info.num_cores * info.num_subcores
    rows_w = B // nw
    per_w = rows_w * L
    half_rows = rows_w // 2
    half = half_rows * L
    assert rows_w * nw == B and half % _LANES == 0

    mesh = plsc.VectorSubcoreMesh(core_axis_name="c", subcore_axis_name="s")

    @functools.partial(
        pl.kernel,
        mesh=mesh,
        compiler_params=pltpu.CompilerParams(needs_layout_passes=False),
        out_type=jax.ShapeDtypeStruct((B, L), jnp.float32),
        scratch_types=[
            pltpu.VMEM((rows_w, L), jnp.int32),
            pltpu.VMEM((rows_w, L), jnp.float32),
            pltpu.VMEM((V, 1), jnp.float32),
            pltpu.SemaphoreType.DMA,
            pltpu.SemaphoreType.DMA,
            pltpu.SemaphoreType.DMA,
            pltpu.SemaphoreType.DMA,
        ],
    )
    def lookup(idx_hbm, tab_hbm, out_hbm, idx_v, out_v, tab_v, si0, si1, so0, so1):
        wid = lax.axis_index("s") * info.num_cores + lax.axis_index("c")
        r0 = wid * rows_w
        in0 = pltpu.async_copy(
            idx_hbm.at[pl.ds(r0, half_rows)], idx_v.at[pl.ds(0, half_rows)], si0)
        in1 = pltpu.async_copy(
            idx_hbm.at[pl.ds(r0 + half_rows, half_rows)],
            idx_v.at[pl.ds(half_rows, half_rows)], si1)
        pltpu.sync_copy(tab_hbm, tab_v)
        lane = lax.iota(jnp.int32, _LANES)
        zero = jnp.zeros((_LANES,), jnp.int32)

        def gather_vec(off):
            o = off + lane
            r = o // L
            c = o - r * L
            iv = plsc.load_gather(idx_v, [r, c])
            vals = plsc.load_gather(tab_v, [iv, zero])
            plsc.store_scatter(out_v, [r, c], vals)

        in0.wait()

        @plsc.parallel_loop(0, half, step=_LANES, unroll=8)
        def body0(off):
            gather_vec(off)

        out0 = pltpu.async_copy(
            out_v.at[pl.ds(0, half_rows)], out_hbm.at[pl.ds(r0, half_rows)], so0)
        in1.wait()

        @plsc.parallel_loop(half, per_w, step=_LANES, unroll=8)
        def body1(off):
            gather_vec(off)

        out1 = pltpu.async_copy(
            out_v.at[pl.ds(half_rows, half_rows)],
            out_hbm.at[pl.ds(r0 + half_rows, half_rows)], so1)
        out0.wait()
        out1.wait()

    return lookup(input, table)


# speedup vs baseline: 1.0228x; 1.0228x over previous
import functools

import jax
import jax.numpy as jnp
from jax import lax
from jax.experimental import pallas as pl
from jax.experimental.pallas import tpu as pltpu
from jax.experimental.pallas import tpu_sc as plsc

_LANES = 16


def kernel(input, table):
    B, L = input.shape
    V = table.shape[0]
    info = plsc.get_sparse_core_info()
    nw = info.num_cores * info.num_subcores
    rows_w = B // nw
    per_w = rows_w * L
    half_rows = rows_w // 2
    half = half_rows * L
    assert rows_w * nw == B and half % _LANES == 0

    mesh = plsc.VectorSubcoreMesh(core_axis_name="c", subcore_axis_name="s")

    @functools.partial(
        pl.kernel,
        mesh=mesh,
        compiler_params=pltpu.CompilerParams(needs_layout_passes=False),
        out_type=jax.ShapeDtypeStruct((B, L), jnp.float32),
        scratch_types=[
            pltpu.VMEM((rows_w, L), jnp.int32),
            pltpu.VMEM((rows_w, L), jnp.float32),
            pltpu.VMEM((V, 1), jnp.float32),
            pltpu.SemaphoreType.DMA,
            pltpu.SemaphoreType.DMA,
            pltpu.SemaphoreType.DMA,
            pltpu.SemaphoreType.DMA,
        ],
    )
    def lookup(idx_hbm, tab_hbm, out_hbm, idx_v, out_v, tab_v, si0, si1, so0, so1):
        wid = lax.axis_index("s") * info.num_cores + lax.axis_index("c")
        r0 = wid * rows_w
        in0 = pltpu.async_copy(
            idx_hbm.at[pl.ds(r0, half_rows)], idx_v.at[pl.ds(0, half_rows)], si0)
        in1 = pltpu.async_copy(
            idx_hbm.at[pl.ds(r0 + half_rows, half_rows)],
            idx_v.at[pl.ds(half_rows, half_rows)], si1)
        pltpu.sync_copy(tab_hbm, tab_v)
        zero = jnp.zeros((_LANES,), jnp.int32)
        col_starts = [k * _LANES for k in range(L // _LANES)] + [L - _LANES]

        def gather_row(r):
            for c0 in col_starts:
                iv = idx_v[r, pl.ds(c0, _LANES)]
                out_v[r, pl.ds(c0, _LANES)] = plsc.load_gather(tab_v, [iv, zero])

        in0.wait()

        @plsc.parallel_loop(0, half_rows, step=1, unroll=2)
        def body0(r):
            gather_row(r)

        out0 = pltpu.async_copy(
            out_v.at[pl.ds(0, half_rows)], out_hbm.at[pl.ds(r0, half_rows)], so0)
        in1.wait()

        @plsc.parallel_loop(half_rows, rows_w, step=1, unroll=2)
        def body1(r):
            gather_row(r)

        out1 = pltpu.async_copy(
            out_v.at[pl.ds(half_rows, half_rows)],
            out_hbm.at[pl.ds(r0 + half_rows, half_rows)], so1)
        out0.wait()
        out1.wait()

    return lookup(input, table)


# speedup vs baseline: 1.6236x; 1.5874x over previous
import functools

import jax
import jax.numpy as jnp
from jax import lax
from jax.experimental import pallas as pl
from jax.experimental.pallas import tpu as pltpu
from jax.experimental.pallas import tpu_sc as plsc

_LANES = 16


def kernel(input, table):
    B, L = input.shape
    V = table.shape[0]
    info = plsc.get_sparse_core_info()
    nw = info.num_cores * info.num_subcores
    rows_w = B // nw
    per_w = rows_w * L
    half_rows = rows_w // 2
    half = half_rows * L
    assert rows_w * nw == B and half % _LANES == 0

    mesh = plsc.VectorSubcoreMesh(core_axis_name="c", subcore_axis_name="s")

    @functools.partial(
        pl.kernel,
        mesh=mesh,
        compiler_params=pltpu.CompilerParams(needs_layout_passes=False),
        out_type=jax.ShapeDtypeStruct((B, L), jnp.float32),
        scratch_types=[
            pltpu.VMEM((rows_w, L), jnp.int32),
            pltpu.VMEM((rows_w, L), jnp.float32),
            pltpu.VMEM((V,), jnp.float32),
            pltpu.SemaphoreType.DMA,
            pltpu.SemaphoreType.DMA,
            pltpu.SemaphoreType.DMA,
            pltpu.SemaphoreType.DMA,
        ],
    )
    def lookup(idx_hbm, tab_hbm, out_hbm, idx_v, out_v, tab_v, si0, si1, so0, so1):
        wid = lax.axis_index("s") * info.num_cores + lax.axis_index("c")
        r0 = wid * rows_w
        in0 = pltpu.async_copy(
            idx_hbm.at[pl.ds(r0, half_rows)], idx_v.at[pl.ds(0, half_rows)], si0)
        in1 = pltpu.async_copy(
            idx_hbm.at[pl.ds(r0 + half_rows, half_rows)],
            idx_v.at[pl.ds(half_rows, half_rows)], si1)
        pltpu.sync_copy(tab_hbm, tab_v)
        col_starts = [k * _LANES for k in range(L // _LANES)] + [L - _LANES]

        def gather_row(r):
            for c0 in col_starts:
                iv = idx_v[r, pl.ds(c0, _LANES)]
                out_v[r, pl.ds(c0, _LANES)] = plsc.load_gather(tab_v, [iv])

        in0.wait()

        @plsc.parallel_loop(0, half_rows, step=1, unroll=2)
        def body0(r):
            gather_row(r)

        out0 = pltpu.async_copy(
            out_v.at[pl.ds(0, half_rows)], out_hbm.at[pl.ds(r0, half_rows)], so0)
        in1.wait()

        @plsc.parallel_loop(half_rows, rows_w, step=1, unroll=2)
        def body1(r):
            gather_row(r)

        out1 = pltpu.async_copy(
            out_v.at[pl.ds(half_rows, half_rows)],
            out_hbm.at[pl.ds(r0 + half_rows, half_rows)], so1)
        out0.wait()
        out1.wait()

    return lookup(input, table.reshape(V))
